# baseline (speedup 1.0000x reference)
import functools

import jax
import jax.numpy as jnp
from jax import lax
from jax.experimental import pallas as pl
from jax.experimental.pallas import tpu as pltpu
from jax.experimental.pallas import tpu_sc as plsc

B = 2048
NPM = 40
N = B * NPM
E = N * 4
NS = 16
RT = N // NS
EQ = E // 128
QT = EQ // NS
QC = 8
NCH = QT // QC

@functools.cache
def _sc_mesh():
    return plsc.VectorSubcoreMesh(
        core_axis_name="c", subcore_axis_name="s", num_cores=2, num_subcores=NS)


def _sc_deg(dstq, ones, zeros):
    @functools.partial(
        pl.kernel,
        out_type=jax.ShapeDtypeStruct((2, N, 16), jnp.float32),
        mesh=_sc_mesh(),
        compiler_params=pltpu.CompilerParams(use_tc_tiling_on_sc=False),
        scratch_types=[
            pltpu.VMEM_SHARED((N, 16), jnp.float32),
            pltpu.VMEM((QC, 128), jnp.int32),
            pltpu.VMEM((QC, 128, 16), jnp.float32),
            pltpu.SemaphoreType.DMA,
        ],
    )
    def k(dstq_h, ones_h, zeros_h, out_h, acc, idx_d, ones_v, sem):
        side = lax.axis_index("c")
        t = lax.axis_index("s")
        pltpu.sync_copy(ones_h, ones_v)
        pltpu.sync_copy(zeros_h.at[pl.ds(t * RT, RT)], acc.at[pl.ds(t * RT, RT)])
        plsc.subcore_barrier()

        def chunk(j, _):
            qb = t * QT + j * QC
            pltpu.sync_copy(dstq_h.at[side, pl.ds(qb, QC)], idx_d)
            descs = [
                pltpu.async_copy(ones_v.at[q], acc.at[idx_d.at[q]], sem, add=True)
                for q in range(QC)
            ]
            for d in descs:
                d.wait()
            return 0

        lax.fori_loop(0, NCH, chunk, 0)
        plsc.subcore_barrier()
        pltpu.sync_copy(acc.at[pl.ds(t * RT, RT)], out_h.at[side, pl.ds(t * RT, RT)])

    return k(dstq, ones, zeros)


def _sc_agg(y_b, srcq, dstq, cb_total):
    @functools.partial(
        pl.kernel,
        out_type=jax.ShapeDtypeStruct((2, cb_total, N, 16), jnp.float32),
        mesh=_sc_mesh(),
        compiler_params=pltpu.CompilerParams(use_tc_tiling_on_sc=False),
        scratch_types=[
            pltpu.VMEM_SHARED((N, 16), jnp.float32),
            pltpu.VMEM((QC, 128), jnp.int32),
            pltpu.VMEM((QC, 128), jnp.int32),
            pltpu.VMEM((QC, 128, 16), jnp.float32),
            pltpu.SemaphoreType.DMA,
            pltpu.SemaphoreType.DMA,
        ],
    )
    def k(y_h, srcq_h, dstq_h, out_h, acc, idx_s, idx_d, rows, gsem, ssem):
        side = lax.axis_index("c")
        t = lax.axis_index("s")
        for cb in range(cb_total):
            pltpu.sync_copy(y_h.at[side, cb, pl.ds(t * RT, RT)],
                            acc.at[pl.ds(t * RT, RT)])
            plsc.subcore_barrier()

            def chunk(j, _):
                qb = t * QT + j * QC
                pltpu.sync_copy(srcq_h.at[side, pl.ds(qb, QC)], idx_s)
                pltpu.sync_copy(dstq_h.at[side, pl.ds(qb, QC)], idx_d)
                gd = [
                    pltpu.async_copy(y_h.at[side, cb].at[idx_s.at[q]],
                                     rows.at[q], gsem)
                    for q in range(QC)
                ]
                for d in gd:
                    d.wait()
                sd = [
                    pltpu.async_copy(rows.at[q], acc.at[idx_d.at[q]], ssem,
                                     add=True)
                    for q in range(QC)
                ]
                for d in sd:
                    d.wait()
                return 0

            lax.fori_loop(0, NCH, chunk, 0)
            plsc.subcore_barrier()
            pltpu.sync_copy(acc.at[pl.ds(t * RT, RT)],
                            out_h.at[side, cb, pl.ds(t * RT, RT)])

    return k(y_b, srcq, dstq)


_R = 512
_NB = N // _R


def _tc_prep(x_pad, deg_b):
    def body(x_ref, d_ref, y_ref, dis_ref):
        deg = d_ref[0, :, 0:1]
        dis = lax.rsqrt(deg + 1.0)
        y = x_ref[0] * dis
        dis_ref[0] = dis
        for c in range(5):
            y_ref[0, c] = y[:, c * 16:(c + 1) * 16]

    return pl.pallas_call(
        body,
        grid=(2, _NB),
        in_specs=[
            pl.BlockSpec((1, _R, 80), lambda s, i: (s, i, 0)),
            pl.BlockSpec((1, _R, 16), lambda s, i: (s, i, 0)),
        ],
        out_specs=[
            pl.BlockSpec((1, 5, _R, 16), lambda s, i: (s, 0, i, 0)),
            pl.BlockSpec((1, _R, 1), lambda s, i: (s, i, 0)),
        ],
        out_shape=[
            jax.ShapeDtypeStruct((2, 5, N, 16), jnp.float32),
            jax.ShapeDtypeStruct((2, N, 1), jnp.float32),
        ],
    )(x_pad, deg_b)


def _tc_mm1(s0_b, dis, W0p, b0, W1):
    def body(s_ref, dis_ref, w0_ref, b0_ref, w1_ref, y_ref):
        s0 = jnp.concatenate([s_ref[0, c] for c in range(5)], axis=1)
        dis = dis_ref[0]
        x1 = jnp.maximum(
            jnp.dot(s0 * dis, w0_ref[...], preferred_element_type=jnp.float32)
            + b0_ref[0], 0.0)
        h = jnp.dot(x1, w1_ref[...], preferred_element_type=jnp.float32)
        y1 = h * dis
        for c in range(8):
            y_ref[0, c] = y1[:, c * 16:(c + 1) * 16]

    return pl.pallas_call(
        body,
        grid=(2, _NB),
        in_specs=[
            pl.BlockSpec((1, 5, _R, 16), lambda s, i: (s, 0, i, 0)),
            pl.BlockSpec((1, _R, 1), lambda s, i: (s, i, 0)),
            pl.BlockSpec((80, 256), lambda s, i: (0, 0)),
            pl.BlockSpec((1, 256), lambda s, i: (0, 0)),
            pl.BlockSpec((256, 128), lambda s, i: (0, 0)),
        ],
        out_specs=pl.BlockSpec((1, 8, _R, 16), lambda s, i: (s, 0, i, 0)),
        out_shape=jax.ShapeDtypeStruct((2, 8, N, 16), jnp.float32),
    )(s0_b, dis, W0p, b0, W1)


def _tc_act(s1_b, dis, b1):
    def body(s_ref, dis_ref, b1_ref, y_ref):
        s1 = jnp.concatenate([s_ref[0, c] for c in range(8)], axis=1)
        dis = dis_ref[0]
        x2 = jnp.maximum(s1 * dis + b1_ref[0], 0.0)
        y2 = x2 * dis
        for c in range(8):
            y_ref[0, c] = y2[:, c * 16:(c + 1) * 16]

    return pl.pallas_call(
        body,
        grid=(2, _NB),
        in_specs=[
            pl.BlockSpec((1, 8, _R, 16), lambda s, i: (s, 0, i, 0)),
            pl.BlockSpec((1, _R, 1), lambda s, i: (s, i, 0)),
            pl.BlockSpec((1, 128), lambda s, i: (0, 0)),
        ],
        out_specs=pl.BlockSpec((1, 8, _R, 16), lambda s, i: (s, 0, i, 0)),
        out_shape=jax.ShapeDtypeStruct((2, 8, N, 16), jnp.float32),
    )(s1_b, dis, b1)


_RM = 640
_NM = N // _RM


def _tc_head(s2_b, dis, W2, b2):
    def body(s_ref, dis_ref, w2_ref, b2_ref, r_ref):
        s2 = jnp.concatenate([s_ref[0, c] for c in range(8)], axis=1)
        dis = dis_ref[0]
        x3 = jnp.maximum(
            jnp.dot(s2 * dis, w2_ref[...], preferred_element_type=jnp.float32)
            + b2_ref[0], 0.0)
        r_ref[0] = jnp.max(x3.reshape(_RM // NPM, NPM, 400), axis=1)

    return pl.pallas_call(
        body,
        grid=(2, _NM),
        in_specs=[
            pl.BlockSpec((1, 8, _RM, 16), lambda s, i: (s, 0, i, 0)),
            pl.BlockSpec((1, _RM, 1), lambda s, i: (s, i, 0)),
            pl.BlockSpec((128, 400), lambda s, i: (0, 0)),
            pl.BlockSpec((1, 400), lambda s, i: (0, 0)),
        ],
        out_specs=pl.BlockSpec((1, _RM // NPM, 400), lambda s, i: (s, i, 0)),
        out_shape=jax.ShapeDtypeStruct((2, B, 400), jnp.float32),
    )(s2_b, dis, W2, b2)


_RB = 256


def _tc_final(ctx, feat, cW0, cb0, cW1, cb1, cW2, cb2,
              dW0, db0, dW1, db1, fA, fB, fC, fb0, fW1, fb1, fW2, fb2):
    def body(c_ref, f_ref, cw0, cb0r, cw1, cb1r, cw2, cb2r,
             dw0, db0r, dw1, db1r, fa, fb, fcr, fb0r, fw1, fb1r, fw2, fb2r,
             o_ref):
        c = c_ref[...]
        nrm = jnp.sqrt(jnp.sum(c * c, axis=1, keepdims=True))
        c = c / jnp.maximum(nrm, 1e-12)
        m = jnp.maximum(jnp.dot(c, cw0[...], preferred_element_type=jnp.float32) + cb0r[0], 0.0)
        m = jnp.maximum(jnp.dot(m, cw1[...], preferred_element_type=jnp.float32) + cb1r[0], 0.0)
        m = jnp.dot(m, cw2[...], preferred_element_type=jnp.float32) + cb2r[0]

        def dmlp(x):
            h = jnp.maximum(jnp.dot(x, dw0[...], preferred_element_type=jnp.float32) + db0r[0], 0.0)
            return jnp.dot(h, dw1[...], preferred_element_type=jnp.float32) + db1r[0]

        dl = dmlp(f_ref[0])
        dr = dmlp(f_ref[1])
        z = jnp.maximum(
            jnp.dot(m, fa[...], preferred_element_type=jnp.float32)
            + jnp.dot(dl, fb[...], preferred_element_type=jnp.float32)
            + jnp.dot(dr, fcr[...], preferred_element_type=jnp.float32)
            + fb0r[0], 0.0)
        z = jnp.maximum(jnp.dot(z, fw1[...], preferred_element_type=jnp.float32) + fb1r[0], 0.0)
        o_ref[...] = jnp.dot(z, fw2[...], preferred_element_type=jnp.float32) + fb2r[0]

    full = lambda a, b: pl.BlockSpec((a, b), lambda i: (0, 0))
    return pl.pallas_call(
        body,
        grid=(B // _RB,),
        in_specs=[
            pl.BlockSpec((_RB, 288), lambda i: (i, 0)),
            pl.BlockSpec((2, _RB, 400), lambda i: (0, i, 0)),
            full(288, 512), full(1, 512), full(512, 256), full(1, 256),
            full(256, 128), full(1, 128),
            full(400, 138), full(1, 138), full(138, 128), full(1, 128),
            full(128, 32), full(128, 32), full(128, 32), full(1, 32),
            full(32, 32), full(1, 32), full(32, 1), full(1, 1),
        ],
        out_specs=pl.BlockSpec((_RB, 1), lambda i: (i, 0)),
        out_shape=jax.ShapeDtypeStruct((B, 1), jnp.float32),
    )(ctx, feat, cW0, cb0, cW1, cb1, cW2, cb2,
      dW0, db0, dW1, db1, fA, fB, fC, fb0, fW1, fb1, fW2, fb2)


def kernel(x_left, x_right, edge_index_left, edge_index_right,
           graph_ids_left, graph_ids_right, context_features,
           ctx_W0, ctx_b0, ctx_W1, ctx_b1, ctx_W2, ctx_b2,
           gcn_W0, gcn_b0, gcn_W1, gcn_b1, gcn_W2, gcn_b2,
           dmlp_W0, dmlp_b0, dmlp_W1, dmlp_b1,
           fc_W0, fc_b0, fc_W1, fc_b1, fc_W2, fc_b2):
    f32 = jnp.float32
    srcq = jnp.stack([edge_index_left[0], edge_index_right[0]]).reshape(2, EQ, 128)
    dstq = jnp.stack([edge_index_left[1], edge_index_right[1]]).reshape(2, EQ, 128)
    x_pad = jnp.stack([
        jnp.pad(x_left, ((0, 0), (0, 11))),
        jnp.pad(x_right, ((0, 0), (0, 11))),
    ])
    W0p = jnp.pad(gcn_W0, ((0, 11), (0, 0)))
    ones = jnp.ones((QC, 128, 16), f32)
    zeros = jnp.zeros((N, 16), f32)

    deg_b = _sc_deg(dstq, ones, zeros)

    y0_b, dis = _tc_prep(x_pad, deg_b)
    s0_b = _sc_agg(y0_b, srcq, dstq, 5)
    y1_b = _tc_mm1(s0_b, dis, W0p, gcn_b0.reshape(1, 256), gcn_W1)
    s1_b = _sc_agg(y1_b, srcq, dstq, 8)
    y2_b = _tc_act(s1_b, dis, gcn_b1.reshape(1, 128))
    s2_b = _sc_agg(y2_b, srcq, dstq, 8)
    feat = _tc_head(s2_b, dis, gcn_W2, gcn_b2.reshape(1, 400))

    out = _tc_final(
        context_features, feat,
        ctx_W0, ctx_b0.reshape(1, 512), ctx_W1, ctx_b1.reshape(1, 256),
        ctx_W2, ctx_b2.reshape(1, 128),
        dmlp_W0, dmlp_b0.reshape(1, 138), dmlp_W1, dmlp_b1.reshape(1, 128),
        fc_W0[:128], fc_W0[128:256], fc_W0[256:384], fc_b0.reshape(1, 32),
        fc_W1, fc_b1.reshape(1, 32), fc_W2, fc_b2.reshape(1, 1),
    )
    return jnp.squeeze(out, axis=-1)
